# Initial kernel scaffold; baseline (speedup 1.0000x reference)
import jax
import jax.numpy as jnp
from jax.experimental import pallas as pl


def kernel(node_fts, means, stds, temp1, temp2):
    raise NotImplementedError("write your pallas kernel here")


# speedup vs baseline: 3.0743x; 3.0743x over previous
import functools

import jax
import jax.numpy as jnp
from jax import lax
from jax.experimental import pallas as pl
from jax.experimental.pallas import tpu as pltpu
from jax.experimental.pallas import tpu_sc as plsc

B = 32
R = 2048
D = 128
V = 65536
K = 50
NB = 4096
CAND = 4352
CANDP = CAND + 16
BIG = 1e30
BIGI = 2**30
L = 16
UNROLL = 8

TILE = 4096
NT = V // TILE


def _stats_body(x_ref, mean_ref, std_ref):
    x = x_ref[0]
    s1 = jnp.sum(x, axis=0)
    s2 = jnp.sum(x * x, axis=0)
    mean = s1 * (1.0 / R)
    var = s2 * (1.0 / R) - mean * mean
    mean_ref[0, 0] = mean
    std_ref[0, 0] = jnp.sqrt(jnp.maximum(var, 0.0))


def _stats(node_fts, interpret=False):
    mq3, sq3 = pl.pallas_call(
        _stats_body,
        grid=(B,),
        in_specs=[pl.BlockSpec((1, R, D), lambda i: (i, 0, 0))],
        out_specs=[pl.BlockSpec((1, 1, D), lambda i: (i, 0, 0))] * 2,
        out_shape=[jax.ShapeDtypeStruct((B, 1, D), jnp.float32)] * 2,
        interpret=interpret,
    )(node_fts)
    return mq3[:, 0], sq3[:, 0]


def _dist_body(m_ref, s_ref, mq_ref, sq_ref, ds_ref, mn_ref, mx_ref):
    i = pl.program_id(0)

    def dpart(tab, q):
        ab = lax.dot_general(q, tab, (((1,), (1,)), ((), ())),
                             preferred_element_type=jnp.float32)
        tn2 = jnp.sum(tab * tab, axis=1)
        qn2 = jnp.sum(q * q, axis=1)
        d2 = qn2[:, None] - 2.0 * ab + tn2[None, :]
        return jnp.sqrt(jnp.maximum(d2, 0.0))

    dst = dpart(m_ref[...], mq_ref[...]) + dpart(s_ref[...], sq_ref[...])
    ds_ref[...] = dst
    tmin = jnp.broadcast_to(jnp.min(dst, axis=1)[:, None], (B, D))
    tmax = jnp.broadcast_to(jnp.max(dst, axis=1)[:, None], (B, D))

    @pl.when(i == 0)
    def _():
        mn_ref[...] = tmin
        mx_ref[...] = tmax

    @pl.when(i > 0)
    def _():
        mn_ref[...] = jnp.minimum(mn_ref[...], tmin)
        mx_ref[...] = jnp.maximum(mx_ref[...], tmax)


def _dist(means, stds, mq, sq, interpret=False):
    return pl.pallas_call(
        _dist_body,
        grid=(NT,),
        in_specs=[
            pl.BlockSpec((TILE, D), lambda i: (i, 0)),
            pl.BlockSpec((TILE, D), lambda i: (i, 0)),
            pl.BlockSpec((B, D), lambda i: (0, 0)),
            pl.BlockSpec((B, D), lambda i: (0, 0)),
        ],
        out_specs=[
            pl.BlockSpec((B, TILE), lambda i: (0, i)),
            pl.BlockSpec((B, D), lambda i: (0, 0)),
            pl.BlockSpec((B, D), lambda i: (0, 0)),
        ],
        out_shape=[
            jax.ShapeDtypeStruct((B, V), jnp.float32),
            jax.ShapeDtypeStruct((B, D), jnp.float32),
            jax.ShapeDtypeStruct((B, D), jnp.float32),
        ],
        interpret=interpret,
    )(means, stds, mq, sq)


def _sel_body(ds_hbm, mn_hbm, mx_hbm, t1_hbm, t2_hbm, rtz_hbm,
              means_hbm, stds_hbm, mq_hbm, sq_hbm, a_hbm, b_hbm,
              dsv, histv, cvalv, cidxv, selv, sidxv, wv, rowsv,
              mnv, mxv, t1sv, t2sv, rtzv, mqv, sqv, av, bv, sem):
    e = lax.axis_index("s") * 2 + lax.axis_index("c")
    iota = lax.iota(jnp.int32, L)

    pltpu.sync_copy(ds_hbm.at[e], dsv)
    pltpu.sync_copy(mn_hbm.at[e], mnv)
    pltpu.sync_copy(mx_hbm.at[e], mxv)
    pltpu.sync_copy(t1_hbm, t1sv)
    pltpu.sync_copy(t2_hbm, t2sv)
    pltpu.sync_copy(rtz_hbm, rtzv)
    pltpu.sync_copy(mq_hbm.at[e], mqv)
    pltpu.sync_copy(sq_hbm.at[e], sqv)

    mn = mnv[...]
    mx = mxv[...]
    scale = jnp.float32(NB) / jnp.maximum(mx - mn, jnp.float32(1e-20))

    def key_of(v):
        return jnp.clip(((v - mn) * scale).astype(jnp.int32), 0, NB - 1)

    def zh(i, _):
        histv[pl.ds(i * L, L)] = jnp.zeros((L,), jnp.int32)
        return 0
    lax.fori_loop(0, NB // L, zh, 0)

    ones_i = jnp.ones((L,), jnp.int32)

    def p1(i, _):
        for u in range(UNROLL):
            v = dsv[pl.ds((i * UNROLL + u) * L, L)]
            plsc.addupdate_scatter(histv, [key_of(v)], ones_i)
        return 0
    lax.fori_loop(0, V // (L * UNROLL), p1, 0)

    def bf(j, carry):
        cnt, bkt = carry
        cs = plsc.cumsum(histv[pl.ds(j * L, L)])
        tot = jnp.max(cs)
        crossed = (cs + cnt) >= K
        ffs = jnp.min(jnp.where(crossed, iota, L))
        found_now = jnp.logical_and(bkt < 0, (cnt + tot) >= K)
        bkt = jnp.where(found_now, j * L + ffs, bkt)
        return (cnt + tot, bkt)
    _, bkt = lax.fori_loop(0, NB // L, bf, (jnp.int32(0), jnp.int32(-1)))
    bktv = jnp.full((L,), bkt, jnp.int32)

    def p3(i, cnt):
        for u in range(UNROLL):
            base = (i * UNROLL + u) * L
            v = dsv[pl.ds(base, L)]
            msk = key_of(v) <= bktv
            inc = plsc.cumsum(jnp.where(msk, 1, 0))
            pos = cnt + inc - 1
            ok = jnp.logical_and(msk, pos < CAND)
            plsc.store_scatter(cvalv, [pos], v, mask=ok)
            plsc.store_scatter(cidxv, [pos], base + iota, mask=ok)
            cnt = jnp.minimum(cnt + jnp.max(inc), jnp.int32(CAND))
        return cnt
    cnt = lax.fori_loop(0, V // (L * UNROLL), p3, jnp.int32(0))

    plsc.store_scatter(cvalv, [cnt + iota], jnp.full((L,), BIG, jnp.float32))
    plsc.store_scatter(cidxv, [cnt + iota], jnp.full((L,), BIGI, jnp.int32))
    nch = (cnt + (L - 1)) >> 4

    selv[pl.ds(48, L)] = jnp.full((L,), BIG, jnp.float32)
    sidxv[pl.ds(48, L)] = jnp.zeros((L,), jnp.int32)

    lane0 = iota == 0

    def sel_round(k, _):
        def scan_chunk(j, c):
            bvv, bi, bp = c
            v = cvalv[pl.ds(j * L, L)]
            ix = cidxv[pl.ds(j * L, L)]
            p = j * L + iota
            better = jnp.logical_or(
                v < bvv, jnp.logical_and(v == bvv, ix < bi))
            return (jnp.where(better, v, bvv),
                    jnp.where(better, ix, bi),
                    jnp.where(better, p, bp))
        bvv, bi, bp = lax.fori_loop(
            0, nch, scan_chunk,
            (jnp.full((L,), BIG, jnp.float32),
             jnp.full((L,), BIGI, jnp.int32),
             jnp.full((L,), BIGI, jnp.int32)))
        m = jnp.min(bvv)
        eq = bvv == m
        imin = jnp.min(jnp.where(eq, bi, BIGI))
        lane = jnp.logical_and(eq, bi == imin)
        p_m = jnp.min(jnp.where(lane, bp, BIGI))
        kv = jnp.full((L,), k, jnp.int32)
        plsc.store_scatter(selv, [kv], jnp.full((L,), m, jnp.float32),
                           mask=lane0)
        plsc.store_scatter(sidxv, [kv], jnp.full((L,), imin, jnp.int32),
                           mask=lane0)
        plsc.store_scatter(cvalv, [jnp.full((L,), p_m, jnp.int32)],
                           jnp.full((L,), BIG, jnp.float32), mask=lane0)
        return 0
    lax.fori_loop(0, K, sel_round, 0)

    t1 = t1sv[...]
    t2 = t2sv[...]
    rtz = rtzv[...]
    lf = 1.0 / (1.0 + jnp.exp(-t2))

    svals = [selv[pl.ds(c * L, L)] for c in range(4)]
    masks = [(c * L + iota) < K for c in range(4)]
    ss = [jnp.where(mk, jnp.exp(-t1 * sv), -BIG)
          for sv, mk in zip(svals, masks)]
    smax = jnp.max(jnp.maximum(jnp.maximum(ss[0], ss[1]),
                               jnp.maximum(ss[2], ss[3])))
    es = [jnp.where(mk, jnp.exp(s - smax), jnp.float32(0.0))
          for s, mk in zip(ss, masks)]
    den = jnp.sum(es[0]) + jnp.sum(es[1]) + jnp.sum(es[2]) + jnp.sum(es[3])
    inv_den = jnp.ones((L,), jnp.float32) / jnp.full((L,), den, jnp.float32)
    for c in range(4):
        wv[pl.ds(c * L, L)] = es[c] * inv_den

    def wsum():
        accs = [jnp.zeros((L,), jnp.float32) for _ in range(D // L)]
        for r in range(64):
            wr = plsc.load_gather(wv, [rtz + r])
            for c in range(D // L):
                accs[c] = accs[c] + wr * rowsv[r, pl.ds(c * L, L)]
        return accs

    pltpu.async_copy(means_hbm.at[sidxv], rowsv, sem).wait()
    mg = wsum()
    pltpu.async_copy(stds_hbm.at[sidxv], rowsv, sem).wait()
    sg = wsum()

    for c in range(D // L):
        mqc = mqv[pl.ds(c * L, L)]
        sqc = sqv[pl.ds(c * L, L)]
        mf = lf * mg[c] + (1.0 - lf) * mqc
        sf = lf * sg[c] + (1.0 - lf) * sqc
        ac = sf / sqc
        av[pl.ds(c * L, L)] = ac
        bv[pl.ds(c * L, L)] = mf - ac * mqc

    pltpu.sync_copy(av, a_hbm.at[e])
    pltpu.sync_copy(bv, b_hbm.at[e])


def _select(ds, mn, mx, t1v, t2v, rtz, means, stds, mq, sq):
    mesh = plsc.VectorSubcoreMesh(core_axis_name="c", subcore_axis_name="s")
    f = functools.partial(
        pl.kernel,
        out_type=[jax.ShapeDtypeStruct((B, D), jnp.float32)] * 2,
        mesh=mesh,
        compiler_params=pltpu.CompilerParams(needs_layout_passes=False),
        scratch_types=[
            pltpu.VMEM((V,), jnp.float32),
            pltpu.VMEM((NB,), jnp.int32),
            pltpu.VMEM((CANDP,), jnp.float32),
            pltpu.VMEM((CANDP,), jnp.int32),
            pltpu.VMEM((64,), jnp.float32),
            pltpu.VMEM((64,), jnp.int32),
            pltpu.VMEM((64,), jnp.float32),
            pltpu.VMEM((64, D), jnp.float32),
            pltpu.VMEM((L,), jnp.float32),
            pltpu.VMEM((L,), jnp.float32),
            pltpu.VMEM((L,), jnp.float32),
            pltpu.VMEM((L,), jnp.float32),
            pltpu.VMEM((L,), jnp.int32),
            pltpu.VMEM((D,), jnp.float32),
            pltpu.VMEM((D,), jnp.float32),
            pltpu.VMEM((D,), jnp.float32),
            pltpu.VMEM((D,), jnp.float32),
            pltpu.SemaphoreType.DMA,
        ],
    )(_sel_body)
    return f(ds, mn, mx, t1v, t2v, rtz, means, stds, mq, sq)


def _final_body(x_ref, a_ref, b_ref, o_ref):
    o_ref[...] = x_ref[...] * a_ref[...] + b_ref[...]


def _final(node_fts, a, b, interpret=False):
    return pl.pallas_call(
        _final_body,
        grid=(B,),
        in_specs=[
            pl.BlockSpec((1, R, D), lambda i: (i, 0, 0)),
            pl.BlockSpec((1, 1, D), lambda i: (i, 0, 0)),
            pl.BlockSpec((1, 1, D), lambda i: (i, 0, 0)),
        ],
        out_specs=pl.BlockSpec((1, R, D), lambda i: (i, 0, 0)),
        out_shape=jax.ShapeDtypeStruct((B, R, D), jnp.float32),
        interpret=interpret,
    )(node_fts, a.reshape(B, 1, D), b.reshape(B, 1, D))


def kernel(node_fts, means, stds, temp1, temp2):
    mq, sq = _stats(node_fts)
    ds, mn, mx = _dist(means, stds, mq, sq)
    t1v = jnp.full((L,), temp1, jnp.float32)
    t2v = jnp.full((L,), temp2, jnp.float32)
    rtz = jnp.zeros((L,), jnp.int32)
    a, b = _select(ds, mn[:, :L], mx[:, :L], t1v, t2v, rtz,
                   means, stds, mq, sq)
    return _final(node_fts, a, b)


# speedup vs baseline: 4.8609x; 1.5811x over previous
import functools

import jax
import jax.numpy as jnp
from jax import lax
from jax.experimental import pallas as pl
from jax.experimental.pallas import tpu as pltpu
from jax.experimental.pallas import tpu_sc as plsc

B = 32
R = 2048
D = 128
V = 65536
K = 50
NB = 4096
STRIDE = 256
NREG = 16 * STRIDE
FRAC = 0.15
BIG = 1e30
BIGI = 2**30
L = 16
UNROLL = 8

TILE = 4096
NT = V // TILE


def _stats_body(x_ref, mean_ref, std_ref):
    x = x_ref[0]
    s1 = jnp.sum(x, axis=0)
    s2 = jnp.sum(x * x, axis=0)
    mean = s1 * (1.0 / R)
    var = s2 * (1.0 / R) - mean * mean
    mean_ref[0, 0] = mean
    std_ref[0, 0] = jnp.sqrt(jnp.maximum(var, 0.0))


def _stats(node_fts, interpret=False):
    mq3, sq3 = pl.pallas_call(
        _stats_body,
        grid=(B,),
        in_specs=[pl.BlockSpec((1, R, D), lambda i: (i, 0, 0))],
        out_specs=[pl.BlockSpec((1, 1, D), lambda i: (i, 0, 0))] * 2,
        out_shape=[jax.ShapeDtypeStruct((B, 1, D), jnp.float32)] * 2,
        interpret=interpret,
    )(node_fts)
    return mq3[:, 0], sq3[:, 0]


def _dist_body(m_ref, s_ref, mq_ref, sq_ref, ds_ref, mn_ref, mx_ref):
    i = pl.program_id(0)

    def dpart(tab, q):
        ab = lax.dot_general(q, tab, (((1,), (1,)), ((), ())),
                             preferred_element_type=jnp.float32)
        tn2 = jnp.sum(tab * tab, axis=1)
        qn2 = jnp.sum(q * q, axis=1)
        d2 = qn2[:, None] - 2.0 * ab + tn2[None, :]
        return jnp.sqrt(jnp.maximum(d2, 0.0))

    dst = dpart(m_ref[...], mq_ref[...]) + dpart(s_ref[...], sq_ref[...])
    ds_ref[...] = dst
    tmin = jnp.broadcast_to(jnp.min(dst, axis=1)[:, None], (B, D))
    tmax = jnp.broadcast_to(jnp.max(dst, axis=1)[:, None], (B, D))

    @pl.when(i == 0)
    def _():
        mn_ref[...] = tmin
        mx_ref[...] = tmax

    @pl.when(i > 0)
    def _():
        mn_ref[...] = jnp.minimum(mn_ref[...], tmin)
        mx_ref[...] = jnp.maximum(mx_ref[...], tmax)


def _dist(means, stds, mq, sq, interpret=False):
    return pl.pallas_call(
        _dist_body,
        grid=(NT,),
        in_specs=[
            pl.BlockSpec((TILE, D), lambda i: (i, 0)),
            pl.BlockSpec((TILE, D), lambda i: (i, 0)),
            pl.BlockSpec((B, D), lambda i: (0, 0)),
            pl.BlockSpec((B, D), lambda i: (0, 0)),
        ],
        out_specs=[
            pl.BlockSpec((B, TILE), lambda i: (0, i)),
            pl.BlockSpec((B, D), lambda i: (0, 0)),
            pl.BlockSpec((B, D), lambda i: (0, 0)),
        ],
        out_shape=[
            jax.ShapeDtypeStruct((B, V), jnp.float32),
            jax.ShapeDtypeStruct((B, D), jnp.float32),
            jax.ShapeDtypeStruct((B, D), jnp.float32),
        ],
        interpret=interpret,
    )(means, stds, mq, sq)


def _sel_body(ds_hbm, mn_hbm, mx_hbm, t1_hbm, t2_hbm, rtz_hbm,
              means_hbm, stds_hbm, mq_hbm, sq_hbm, a_hbm, b_hbm,
              dsv, histv, cvalv, cidxv, cntsv, selv, sidxv, wv, rowsv,
              mnv, mxv, t1sv, t2sv, rtzv, mqv, sqv, av, bv, sem):
    e = lax.axis_index("s") * 2 + lax.axis_index("c")
    iota = lax.iota(jnp.int32, L)

    pltpu.sync_copy(ds_hbm.at[e], dsv)
    pltpu.sync_copy(mn_hbm.at[e], mnv)
    pltpu.sync_copy(mx_hbm.at[e], mxv)
    pltpu.sync_copy(t1_hbm, t1sv)
    pltpu.sync_copy(t2_hbm, t2sv)
    pltpu.sync_copy(rtz_hbm, rtzv)
    pltpu.sync_copy(mq_hbm.at[e], mqv)
    pltpu.sync_copy(sq_hbm.at[e], sqv)

    mn = mnv[...]
    mx = mxv[...]
    scale = jnp.float32(NB) / jnp.maximum(mx - mn, jnp.float32(1e-20))

    def key_of(v):
        return jnp.clip(((v - mn) * scale).astype(jnp.int32), 0, NB - 1)

    ones_i = jnp.ones((L,), jnp.int32)
    big_f = jnp.full((L,), BIG, jnp.float32)
    lanebase = iota * STRIDE

    def ci(i, _):
        for u in range(UNROLL):
            cvalv[pl.ds((i * UNROLL + u) * L, L)] = big_f
        return 0
    lax.fori_loop(0, NREG // (L * UNROLL), ci, 0)

    def collect(pred):
        def body(i, cnt_v):
            for u in range(UNROLL):
                base = (i * UNROLL + u) * L
                v = dsv[pl.ds(base, L)]
                msk = pred(v)
                pos = lanebase + jnp.minimum(cnt_v, STRIDE - 1)
                plsc.store_scatter(cvalv, [pos], v, mask=msk)
                plsc.store_scatter(cidxv, [pos], base + iota, mask=msk)
                cnt_v = cnt_v + jnp.where(msk, 1, 0)
            return cnt_v
        return lax.fori_loop(0, V // (L * UNROLL), body,
                             jnp.zeros((L,), jnp.int32))

    tspec = mn + (mx - mn) * jnp.float32(FRAC)
    cnt_v0 = collect(lambda v: v < tspec)
    cntsv[...] = cnt_v0
    total = jnp.sum(cnt_v0)
    mmax = jnp.max(cnt_v0)
    need_fb = jnp.logical_or(total < K, mmax >= STRIDE)

    @pl.when(need_fb)
    def _fb():
        def zh(i, _):
            histv[pl.ds(i * L, L)] = jnp.zeros((L,), jnp.int32)
            return 0
        lax.fori_loop(0, NB // L, zh, 0)

        def p1(i, _):
            for u in range(UNROLL):
                v = dsv[pl.ds((i * UNROLL + u) * L, L)]
                plsc.addupdate_scatter(histv, [key_of(v)], ones_i)
            return 0
        lax.fori_loop(0, V // (L * UNROLL), p1, 0)

        def bf(j, carry):
            cnt, bkt = carry
            cs = plsc.cumsum(histv[pl.ds(j * L, L)])
            tot = jnp.max(cs)
            crossed = (cs + cnt) >= K
            ffs = jnp.min(jnp.where(crossed, iota, L))
            found_now = jnp.logical_and(bkt < 0, (cnt + tot) >= K)
            bkt = jnp.where(found_now, j * L + ffs, bkt)
            return (cnt + tot, bkt)
        _, bkt = lax.fori_loop(0, NB // L, bf, (jnp.int32(0), jnp.int32(-1)))
        bktv = jnp.full((L,), bkt, jnp.int32)

        def ci2(i, _):
            for u in range(UNROLL):
                cvalv[pl.ds((i * UNROLL + u) * L, L)] = big_f
            return 0
        lax.fori_loop(0, NREG // (L * UNROLL), ci2, 0)
        cntsv[...] = collect(lambda v: key_of(v) <= bktv)

    cnt_v = cntsv[...]
    nch = jnp.max(jnp.minimum(cnt_v, STRIDE))

    selv[pl.ds(48, L)] = big_f
    sidxv[pl.ds(48, L)] = jnp.zeros((L,), jnp.int32)

    lane0 = iota == 0

    def sel_round(k, _):
        def scan_chunk(c, carry):
            bvv, bi, bp = carry
            pos = lanebase + c
            v = plsc.load_gather(cvalv, [pos])
            ix = plsc.load_gather(cidxv, [pos])
            better = jnp.logical_or(
                v < bvv, jnp.logical_and(v == bvv, ix < bi))
            return (jnp.where(better, v, bvv),
                    jnp.where(better, ix, bi),
                    jnp.where(better, pos, bp))
        bvv, bi, bp = lax.fori_loop(
            0, nch, scan_chunk,
            (jnp.full((L,), BIG, jnp.float32),
             jnp.full((L,), BIGI, jnp.int32),
             jnp.full((L,), BIGI, jnp.int32)))
        m = jnp.min(bvv)
        eq = bvv == m
        imin = jnp.min(jnp.where(eq, bi, BIGI))
        lane = jnp.logical_and(eq, bi == imin)
        p_m = jnp.min(jnp.where(lane, bp, BIGI))
        kv = jnp.full((L,), k, jnp.int32)
        plsc.store_scatter(selv, [kv], jnp.full((L,), m, jnp.float32),
                           mask=lane0)
        plsc.store_scatter(sidxv, [kv], jnp.full((L,), imin, jnp.int32),
                           mask=lane0)
        plsc.store_scatter(cvalv, [jnp.full((L,), p_m, jnp.int32)],
                           jnp.full((L,), BIG, jnp.float32), mask=lane0)
        return 0
    lax.fori_loop(0, K, sel_round, 0)

    t1 = t1sv[...]
    t2 = t2sv[...]
    rtz = rtzv[...]
    lf = 1.0 / (1.0 + jnp.exp(-t2))

    svals = [selv[pl.ds(c * L, L)] for c in range(4)]
    masks = [(c * L + iota) < K for c in range(4)]
    ss = [jnp.where(mk, jnp.exp(-t1 * sv), -BIG)
          for sv, mk in zip(svals, masks)]
    smax = jnp.max(jnp.maximum(jnp.maximum(ss[0], ss[1]),
                               jnp.maximum(ss[2], ss[3])))
    es = [jnp.where(mk, jnp.exp(s - smax), jnp.float32(0.0))
          for s, mk in zip(ss, masks)]
    den = jnp.sum(es[0]) + jnp.sum(es[1]) + jnp.sum(es[2]) + jnp.sum(es[3])
    inv_den = jnp.ones((L,), jnp.float32) / jnp.full((L,), den, jnp.float32)
    for c in range(4):
        wv[pl.ds(c * L, L)] = es[c] * inv_den

    def wsum():
        accs = [jnp.zeros((L,), jnp.float32) for _ in range(D // L)]
        for r in range(64):
            wr = plsc.load_gather(wv, [rtz + r])
            for c in range(D // L):
                accs[c] = accs[c] + wr * rowsv[r, pl.ds(c * L, L)]
        return accs

    pltpu.async_copy(means_hbm.at[sidxv], rowsv, sem).wait()
    mg = wsum()
    pltpu.async_copy(stds_hbm.at[sidxv], rowsv, sem).wait()
    sg = wsum()

    for c in range(D // L):
        mqc = mqv[pl.ds(c * L, L)]
        sqc = sqv[pl.ds(c * L, L)]
        mf = lf * mg[c] + (1.0 - lf) * mqc
        sf = lf * sg[c] + (1.0 - lf) * sqc
        ac = sf / sqc
        av[pl.ds(c * L, L)] = ac
        bv[pl.ds(c * L, L)] = mf - ac * mqc

    pltpu.sync_copy(av, a_hbm.at[e])
    pltpu.sync_copy(bv, b_hbm.at[e])


def _select(ds, mn, mx, t1v, t2v, rtz, means, stds, mq, sq):
    mesh = plsc.VectorSubcoreMesh(core_axis_name="c", subcore_axis_name="s")
    f = functools.partial(
        pl.kernel,
        out_type=[jax.ShapeDtypeStruct((B, D), jnp.float32)] * 2,
        mesh=mesh,
        compiler_params=pltpu.CompilerParams(needs_layout_passes=False),
        scratch_types=[
            pltpu.VMEM((V,), jnp.float32),
            pltpu.VMEM((NB,), jnp.int32),
            pltpu.VMEM((NREG,), jnp.float32),
            pltpu.VMEM((NREG,), jnp.int32),
            pltpu.VMEM((L,), jnp.int32),
            pltpu.VMEM((64,), jnp.float32),
            pltpu.VMEM((64,), jnp.int32),
            pltpu.VMEM((64,), jnp.float32),
            pltpu.VMEM((64, D), jnp.float32),
            pltpu.VMEM((L,), jnp.float32),
            pltpu.VMEM((L,), jnp.float32),
            pltpu.VMEM((L,), jnp.float32),
            pltpu.VMEM((L,), jnp.float32),
            pltpu.VMEM((L,), jnp.int32),
            pltpu.VMEM((D,), jnp.float32),
            pltpu.VMEM((D,), jnp.float32),
            pltpu.VMEM((D,), jnp.float32),
            pltpu.VMEM((D,), jnp.float32),
            pltpu.SemaphoreType.DMA,
        ],
    )(_sel_body)
    return f(ds, mn, mx, t1v, t2v, rtz, means, stds, mq, sq)


def _final_body(x_ref, a_ref, b_ref, o_ref):
    o_ref[...] = x_ref[...] * a_ref[...] + b_ref[...]


def _final(node_fts, a, b, interpret=False):
    return pl.pallas_call(
        _final_body,
        grid=(B,),
        in_specs=[
            pl.BlockSpec((1, R, D), lambda i: (i, 0, 0)),
            pl.BlockSpec((1, 1, D), lambda i: (i, 0, 0)),
            pl.BlockSpec((1, 1, D), lambda i: (i, 0, 0)),
        ],
        out_specs=pl.BlockSpec((1, R, D), lambda i: (i, 0, 0)),
        out_shape=jax.ShapeDtypeStruct((B, R, D), jnp.float32),
        interpret=interpret,
    )(node_fts, a.reshape(B, 1, D), b.reshape(B, 1, D))


def kernel(node_fts, means, stds, temp1, temp2):
    mq, sq = _stats(node_fts)
    ds, mn, mx = _dist(means, stds, mq, sq)
    t1v = jnp.full((L,), temp1, jnp.float32)
    t2v = jnp.full((L,), temp2, jnp.float32)
    rtz = jnp.zeros((L,), jnp.int32)
    a, b = _select(ds, mn[:, :L], mx[:, :L], t1v, t2v, rtz,
                   means, stds, mq, sq)
    return _final(node_fts, a, b)


# speedup vs baseline: 5.4385x; 1.1188x over previous
import functools

import jax
import jax.numpy as jnp
from jax import lax
from jax.experimental import pallas as pl
from jax.experimental.pallas import tpu as pltpu
from jax.experimental.pallas import tpu_sc as plsc

B = 32
R = 2048
D = 128
V = 65536
K = 50
NB = 4096
STRIDE = 256
NREG = 8192
UNROLL_C = 16
FRAC = 0.15
BIG = 1e30
BIGI = 2**30
L = 16
UNROLL = 8

TILE = 4096
NT = V // TILE


SB = 4


def _stats_body(x_ref, mean_ref, std_ref):
    x = x_ref[...]
    s1 = jnp.sum(x, axis=1)
    s2 = jnp.sum(x * x, axis=1)
    mean = s1 * (1.0 / R)
    var = s2 * (1.0 / R) - mean * mean
    mean_ref[...] = mean[:, None, :]
    std_ref[...] = jnp.sqrt(jnp.maximum(var, 0.0))[:, None, :]


def _stats(node_fts, interpret=False):
    mq3, sq3 = pl.pallas_call(
        _stats_body,
        grid=(B // SB,),
        in_specs=[pl.BlockSpec((SB, R, D), lambda i: (i, 0, 0))],
        out_specs=[pl.BlockSpec((SB, 1, D), lambda i: (i, 0, 0))] * 2,
        out_shape=[jax.ShapeDtypeStruct((B, 1, D), jnp.float32)] * 2,
        interpret=interpret,
    )(node_fts)
    return mq3[:, 0], sq3[:, 0]


def _dist_body(m_ref, s_ref, mq_ref, sq_ref, ds_ref, mn_ref, mx_ref):
    i = pl.program_id(0)

    def dpart(tab, q):
        ab = lax.dot_general(q, tab, (((1,), (1,)), ((), ())),
                             preferred_element_type=jnp.float32)
        tn2 = jnp.sum(tab * tab, axis=1)
        qn2 = jnp.sum(q * q, axis=1)
        d2 = qn2[:, None] - 2.0 * ab + tn2[None, :]
        return jnp.sqrt(jnp.maximum(d2, 0.0))

    dst = dpart(m_ref[...], mq_ref[...]) + dpart(s_ref[...], sq_ref[...])
    ds_ref[...] = dst
    tmin = jnp.broadcast_to(jnp.min(dst, axis=1)[:, None], (B, D))
    tmax = jnp.broadcast_to(jnp.max(dst, axis=1)[:, None], (B, D))

    @pl.when(i == 0)
    def _():
        mn_ref[...] = tmin
        mx_ref[...] = tmax

    @pl.when(i > 0)
    def _():
        mn_ref[...] = jnp.minimum(mn_ref[...], tmin)
        mx_ref[...] = jnp.maximum(mx_ref[...], tmax)


def _dist(means, stds, mq, sq, interpret=False):
    return pl.pallas_call(
        _dist_body,
        grid=(NT,),
        in_specs=[
            pl.BlockSpec((TILE, D), lambda i: (i, 0)),
            pl.BlockSpec((TILE, D), lambda i: (i, 0)),
            pl.BlockSpec((B, D), lambda i: (0, 0)),
            pl.BlockSpec((B, D), lambda i: (0, 0)),
        ],
        out_specs=[
            pl.BlockSpec((B, TILE), lambda i: (0, i)),
            pl.BlockSpec((B, D), lambda i: (0, 0)),
            pl.BlockSpec((B, D), lambda i: (0, 0)),
        ],
        out_shape=[
            jax.ShapeDtypeStruct((B, V), jnp.float32),
            jax.ShapeDtypeStruct((B, D), jnp.float32),
            jax.ShapeDtypeStruct((B, D), jnp.float32),
        ],
        interpret=interpret,
    )(means, stds, mq, sq)


def _sel_body(ds_hbm, mn_hbm, mx_hbm, t1_hbm, t2_hbm, rtz_hbm,
              means_hbm, stds_hbm, mq_hbm, sq_hbm, a_hbm, b_hbm,
              dsv, histv, cvalv, cidxv, cntsv, selv, sidxv, wv, rowsv, rows2v,
              mnv, mxv, t1sv, t2sv, rtzv, mqv, sqv, av, bv, sem, sem2):
    e = lax.axis_index("s") * 2 + lax.axis_index("c")
    iota = lax.iota(jnp.int32, L)

    H = V // 2
    cp1 = pltpu.async_copy(ds_hbm.at[e, pl.ds(0, H)], dsv.at[pl.ds(0, H)], sem)
    cp2 = pltpu.async_copy(ds_hbm.at[e, pl.ds(H, H)], dsv.at[pl.ds(H, H)],
                           sem2)
    pltpu.sync_copy(mn_hbm.at[e], mnv)
    pltpu.sync_copy(mx_hbm.at[e], mxv)
    pltpu.sync_copy(t1_hbm, t1sv)
    pltpu.sync_copy(t2_hbm, t2sv)
    pltpu.sync_copy(rtz_hbm, rtzv)
    pltpu.sync_copy(mq_hbm.at[e], mqv)
    pltpu.sync_copy(sq_hbm.at[e], sqv)

    mn = mnv[...]
    mx = mxv[...]
    scale = jnp.float32(NB) / jnp.maximum(mx - mn, jnp.float32(1e-20))

    def key_of(v):
        return jnp.clip(((v - mn) * scale).astype(jnp.int32), 0, NB - 1)

    ones_i = jnp.ones((L,), jnp.int32)
    big_f = jnp.full((L,), BIG, jnp.float32)
    lanebase = iota * STRIDE

    def ci(i, _):
        for u in range(UNROLL):
            cvalv[pl.ds((i * UNROLL + u) * L, L)] = big_f
        return 0
    lax.fori_loop(0, NREG // (L * UNROLL), ci, 0)

    def collect(pred, lo, hi, cnt0):
        def body(i, cnt_v):
            for u in range(UNROLL_C):
                base = lo + (i * UNROLL_C + u) * L
                v = dsv[pl.ds(base, L)]
                msk = pred(v)
                pos = lanebase + cnt_v
                plsc.store_scatter(cvalv, [pos], v, mask=msk)
                plsc.store_scatter(cidxv, [pos], base + iota, mask=msk)
                cnt_v = cnt_v + jnp.where(msk, 1, 0)
            return cnt_v
        return lax.fori_loop(0, (hi - lo) // (L * UNROLL_C), body, cnt0)

    tspec = mn + (mx - mn) * jnp.float32(FRAC)
    zc = jnp.zeros((L,), jnp.int32)
    cp1.wait()
    cnt_h = collect(lambda v: v < tspec, 0, H, zc)
    cp2.wait()
    cnt_v0 = collect(lambda v: v < tspec, H, V, cnt_h)
    cntsv[...] = cnt_v0
    total = jnp.sum(cnt_v0)
    mmax = jnp.max(cnt_v0)
    need_fb = jnp.logical_or(total < K, mmax >= STRIDE)

    @pl.when(need_fb)
    def _fb():
        def zh(i, _):
            histv[pl.ds(i * L, L)] = jnp.zeros((L,), jnp.int32)
            return 0
        lax.fori_loop(0, NB // L, zh, 0)

        def p1(i, _):
            for u in range(UNROLL):
                v = dsv[pl.ds((i * UNROLL + u) * L, L)]
                plsc.addupdate_scatter(histv, [key_of(v)], ones_i)
            return 0
        lax.fori_loop(0, V // (L * UNROLL), p1, 0)

        def bf(j, carry):
            cnt, bkt = carry
            cs = plsc.cumsum(histv[pl.ds(j * L, L)])
            tot = jnp.max(cs)
            crossed = (cs + cnt) >= K
            ffs = jnp.min(jnp.where(crossed, iota, L))
            found_now = jnp.logical_and(bkt < 0, (cnt + tot) >= K)
            bkt = jnp.where(found_now, j * L + ffs, bkt)
            return (cnt + tot, bkt)
        _, bkt = lax.fori_loop(0, NB // L, bf, (jnp.int32(0), jnp.int32(-1)))
        bktv = jnp.full((L,), bkt, jnp.int32)

        def ci2(i, _):
            for u in range(UNROLL):
                cvalv[pl.ds((i * UNROLL + u) * L, L)] = big_f
            return 0
        lax.fori_loop(0, NREG // (L * UNROLL), ci2, 0)
        cntsv[...] = collect(lambda v: key_of(v) <= bktv, 0, V,
                             jnp.zeros((L,), jnp.int32))

    cnt_v = cntsv[...]
    nch = jnp.max(jnp.minimum(cnt_v, STRIDE))

    selv[pl.ds(48, L)] = big_f
    sidxv[pl.ds(48, L)] = jnp.zeros((L,), jnp.int32)

    lane0 = iota == 0

    def sel_round(k, _):
        def scan_chunk(c, carry):
            bvv, bi, bp = carry
            pos = lanebase + c
            v = plsc.load_gather(cvalv, [pos])
            ix = plsc.load_gather(cidxv, [pos])
            better = jnp.logical_or(
                v < bvv, jnp.logical_and(v == bvv, ix < bi))
            return (jnp.where(better, v, bvv),
                    jnp.where(better, ix, bi),
                    jnp.where(better, pos, bp))
        bvv, bi, bp = lax.fori_loop(
            0, nch, scan_chunk,
            (jnp.full((L,), BIG, jnp.float32),
             jnp.full((L,), BIGI, jnp.int32),
             jnp.full((L,), BIGI, jnp.int32)))
        m = jnp.min(bvv)
        eq = bvv == m
        imin = jnp.min(jnp.where(eq, bi, BIGI))
        lane = jnp.logical_and(eq, bi == imin)
        p_m = jnp.min(jnp.where(lane, bp, BIGI))
        kv = jnp.full((L,), k, jnp.int32)
        plsc.store_scatter(selv, [kv], jnp.full((L,), m, jnp.float32),
                           mask=lane0)
        plsc.store_scatter(sidxv, [kv], jnp.full((L,), imin, jnp.int32),
                           mask=lane0)
        plsc.store_scatter(cvalv, [jnp.full((L,), p_m, jnp.int32)],
                           jnp.full((L,), BIG, jnp.float32), mask=lane0)
        return 0
    lax.fori_loop(0, K, sel_round, 0)

    t1 = t1sv[...]
    t2 = t2sv[...]
    rtz = rtzv[...]
    lf = 1.0 / (1.0 + jnp.exp(-t2))

    svals = [selv[pl.ds(c * L, L)] for c in range(4)]
    masks = [(c * L + iota) < K for c in range(4)]
    ss = [jnp.where(mk, jnp.exp(-t1 * sv), -BIG)
          for sv, mk in zip(svals, masks)]
    smax = jnp.max(jnp.maximum(jnp.maximum(ss[0], ss[1]),
                               jnp.maximum(ss[2], ss[3])))
    es = [jnp.where(mk, jnp.exp(s - smax), jnp.float32(0.0))
          for s, mk in zip(ss, masks)]
    den = jnp.sum(es[0]) + jnp.sum(es[1]) + jnp.sum(es[2]) + jnp.sum(es[3])
    inv_den = jnp.ones((L,), jnp.float32) / jnp.full((L,), den, jnp.float32)
    for c in range(4):
        wv[pl.ds(c * L, L)] = es[c] * inv_den

    def wsum(rows):
        accs = [jnp.zeros((L,), jnp.float32) for _ in range(D // L)]
        for r in range(64):
            wr = plsc.load_gather(wv, [rtz + r])
            for c in range(D // L):
                accs[c] = accs[c] + wr * rows[r, pl.ds(c * L, L)]
        return accs

    cpm = pltpu.async_copy(means_hbm.at[sidxv], rowsv, sem)
    cps = pltpu.async_copy(stds_hbm.at[sidxv], rows2v, sem2)
    cpm.wait()
    mg = wsum(rowsv)
    cps.wait()
    sg = wsum(rows2v)

    for c in range(D // L):
        mqc = mqv[pl.ds(c * L, L)]
        sqc = sqv[pl.ds(c * L, L)]
        mf = lf * mg[c] + (1.0 - lf) * mqc
        sf = lf * sg[c] + (1.0 - lf) * sqc
        ac = sf / sqc
        av[pl.ds(c * L, L)] = ac
        bv[pl.ds(c * L, L)] = mf - ac * mqc

    pltpu.sync_copy(av, a_hbm.at[e])
    pltpu.sync_copy(bv, b_hbm.at[e])


def _select(ds, mn, mx, t1v, t2v, rtz, means, stds, mq, sq):
    mesh = plsc.VectorSubcoreMesh(core_axis_name="c", subcore_axis_name="s")
    f = functools.partial(
        pl.kernel,
        out_type=[jax.ShapeDtypeStruct((B, D), jnp.float32)] * 2,
        mesh=mesh,
        compiler_params=pltpu.CompilerParams(needs_layout_passes=False),
        scratch_types=[
            pltpu.VMEM((V,), jnp.float32),
            pltpu.VMEM((NB,), jnp.int32),
            pltpu.VMEM((NREG,), jnp.float32),
            pltpu.VMEM((NREG,), jnp.int32),
            pltpu.VMEM((L,), jnp.int32),
            pltpu.VMEM((64,), jnp.float32),
            pltpu.VMEM((64,), jnp.int32),
            pltpu.VMEM((64,), jnp.float32),
            pltpu.VMEM((64, D), jnp.float32),
            pltpu.VMEM((64, D), jnp.float32),
            pltpu.VMEM((L,), jnp.float32),
            pltpu.VMEM((L,), jnp.float32),
            pltpu.VMEM((L,), jnp.float32),
            pltpu.VMEM((L,), jnp.float32),
            pltpu.VMEM((L,), jnp.int32),
            pltpu.VMEM((D,), jnp.float32),
            pltpu.VMEM((D,), jnp.float32),
            pltpu.VMEM((D,), jnp.float32),
            pltpu.VMEM((D,), jnp.float32),
            pltpu.SemaphoreType.DMA,
            pltpu.SemaphoreType.DMA,
        ],
    )(_sel_body)
    return f(ds, mn, mx, t1v, t2v, rtz, means, stds, mq, sq)


def _final_body(x_ref, a_ref, b_ref, o_ref):
    o_ref[...] = x_ref[...] * a_ref[...] + b_ref[...]


def _final(node_fts, a, b, interpret=False):
    return pl.pallas_call(
        _final_body,
        grid=(B,),
        in_specs=[
            pl.BlockSpec((1, R, D), lambda i: (i, 0, 0)),
            pl.BlockSpec((1, 1, D), lambda i: (i, 0, 0)),
            pl.BlockSpec((1, 1, D), lambda i: (i, 0, 0)),
        ],
        out_specs=pl.BlockSpec((1, R, D), lambda i: (i, 0, 0)),
        out_shape=jax.ShapeDtypeStruct((B, R, D), jnp.float32),
        interpret=interpret,
    )(node_fts, a.reshape(B, 1, D), b.reshape(B, 1, D))


def kernel(node_fts, means, stds, temp1, temp2):
    mq, sq = _stats(node_fts)
    ds, mn, mx = _dist(means, stds, mq, sq)
    t1v = jnp.full((L,), temp1, jnp.float32)
    t2v = jnp.full((L,), temp2, jnp.float32)
    rtz = jnp.zeros((L,), jnp.int32)
    a, b = _select(ds, mn[:, :L], mx[:, :L], t1v, t2v, rtz,
                   means, stds, mq, sq)
    return _final(node_fts, a, b)
